# baseline (speedup 1.0000x reference)
import jax
import jax.numpy as jnp
from jax.experimental import pallas as pl

_B, _N, _C, _T = 2, 200, 3, 1024
_D, _K, _NUM_GC, _PRED = 64, 8, 4, 64
_ROWS = _B * _N
_R = 8


def _leaky(v):
    return jnp.where(v >= 0, v, 0.01 * v)


def _temporal_body(x_ref, *refs):
    w_refs = refs[:7]
    b_refs = refs[7:14]
    out_ref = refs[14]
    h = x_ref[...]
    h = jnp.transpose(h, (0, 2, 1))
    for l in range(7):
        r, tin, cin = h.shape
        tw = tin + 1
        z = jnp.zeros((r, 2, cin), jnp.float32)
        hp = jnp.concatenate([z, h, z], axis=1)
        feat = jnp.concatenate([hp[:, j:j + tw, :] for j in range(4)],
                               axis=-1)
        w = w_refs[l][...]
        out = feat.reshape(r * tw, 4 * cin) @ w
        out = out + b_refs[l][...]
        out = _leaky(out).reshape(r, tw, _D)
        tp = tin // 2
        h = jnp.max(out[:, :2 * tp, :].reshape(r, tp, 2, _D), axis=2)
    out_ref[...] = jnp.mean(h, axis=1)


def _knn_masks(xm, n):
    sq = jnp.sum(xm * xm, axis=1)
    g = jax.lax.dot_general(xm, xm, (((1,), (1,)), ((), ())),
                            preferred_element_type=jnp.float32)
    d2 = sq[:, None] + sq[None, :] - 2.0 * g
    dist = jnp.sqrt(jnp.maximum(d2, 0.0))
    colids = jax.lax.broadcasted_iota(jnp.int32, (n, n), 1)
    masks = []
    for it in range(_K + 1):
        rowmin = jnp.min(dist, axis=1, keepdims=True)
        cand = jnp.where(dist == rowmin, colids, jnp.int32(2**30))
        first = jnp.min(cand, axis=1, keepdims=True)
        oh = colids == first
        if it > 0:
            masks.append(oh.astype(jnp.float32))
        dist = jnp.where(oh, jnp.float32(jnp.inf), dist)
    return masks


def _graph_body(t0_ref, cd_ref, *refs):
    gc = refs[:8 * _NUM_GC]
    site_w, site_b, cls_w1, cls_b1, cls_w2, cls_b2, out_ref = refs[8 * _NUM_GC:]
    x = t0_ref[0]
    cd = cd_ref[0]
    n = _N

    masks_d = _knn_masks(cd, n)
    acc_site = x @ site_w[:_D]
    for l in range(_NUM_GC):
        fw1, fb1, fw2, fb2, dw1, db1, dw2, db2 = gc[8 * l:8 * l + 8]
        masks_f = _knn_masks(x, n)
        w1 = fw1[...]
        base_f = x @ w1[:_D] + fb1[...]
        w1b = w1[_D:]
        w2 = fw2[...]
        b2 = fb2[...]
        xf = None
        for oh in masks_f:
            neigh = oh @ x
            hh = _leaky(base_f + (neigh - x) @ w1b)
            o = hh @ w2 + b2
            xf = o if xf is None else jnp.maximum(xf, o)
        v1 = dw1[...]
        base_d = x @ v1[:_D] + cd @ v1[2 * _D:2 * _D + 3] + db1[...]
        v1b = v1[_D:2 * _D]
        v1d = v1[2 * _D + 3:2 * _D + 6]
        u2 = dw2[...]
        c2 = db2[...]
        xd = None
        for oh in masks_d:
            neigh = oh @ x
            neighc = oh @ cd
            hh = _leaky(base_d + (neigh - x) @ v1b + neighc @ v1d)
            o = hh @ u2 + c2
            xd = o if xd is None else jnp.maximum(xd, o)
        x = xf + xd
        acc_site = acc_site + x @ site_w[_D * (l + 1):_D * (l + 2)]
    site = acc_site + site_b[...]
    pooled = jnp.max(site, axis=0, keepdims=True)
    hh = _leaky(pooled @ cls_w1[...] + cls_b1[...])
    out_ref[...] = (hh @ cls_w2[...] + cls_b2[...])[None]


def _full_spec(shape):
    nd = len(shape)
    return pl.BlockSpec(shape, lambda *_: (0,) * nd)


def kernel(X, coords, params):
    f32 = jnp.float32
    Xr = X.reshape(_ROWS, _C, _T)
    conv_ws, conv_bs = [], []
    for l in range(7):
        w = params['conv_w_%d' % l]
        conv_ws.append(jnp.transpose(w, (2, 1, 0)).reshape(-1, _D))
        conv_bs.append(params['conv_b_%d' % l].reshape(1, _D))
    grid_t = _ROWS // _R
    in_specs = [pl.BlockSpec((_R, _C, _T), lambda i: (i, 0, 0))]
    in_specs += [_full_spec(w.shape) for w in conv_ws]
    in_specs += [_full_spec(b.shape) for b in conv_bs]
    t0 = pl.pallas_call(
        _temporal_body,
        grid=(grid_t,),
        in_specs=in_specs,
        out_specs=pl.BlockSpec((_R, _D), lambda i: (i, 0)),
        out_shape=jax.ShapeDtypeStruct((_ROWS, _D), f32),
    )(Xr, *conv_ws, *conv_bs)
    t0 = t0.reshape(_B, _N, _D)

    gc_wts = []
    for l in range(_NUM_GC):
        gc_wts += [
            params['f_w1_%d' % l], params['f_b1_%d' % l].reshape(1, _D),
            params['f_w2_%d' % l], params['f_b2_%d' % l].reshape(1, _D),
            params['d_w1_%d' % l], params['d_b1_%d' % l].reshape(1, _D),
            params['d_w2_%d' % l], params['d_b2_%d' % l].reshape(1, _D),
        ]
    tail_wts = [
        params['site_w'], params['site_b'].reshape(1, _PRED),
        params['cls_w1'], params['cls_b1'].reshape(1, _PRED),
        params['cls_w2'], params['cls_b2'].reshape(1, 3),
    ]
    in_specs = [
        pl.BlockSpec((1, _N, _D), lambda b: (b, 0, 0)),
        pl.BlockSpec((1, _N, 3), lambda b: (b, 0, 0)),
    ]
    in_specs += [_full_spec(w.shape) for w in gc_wts]
    in_specs += [_full_spec(w.shape) for w in tail_wts]
    out = pl.pallas_call(
        _graph_body,
        grid=(_B,),
        in_specs=in_specs,
        out_specs=pl.BlockSpec((1, 1, 3), lambda b: (b, 0, 0)),
        out_shape=jax.ShapeDtypeStruct((_B, 1, 3), f32),
    )(t0, coords, *gc_wts, *tail_wts)
    return out.reshape(_B, 3)
